# baseline (speedup 1.0000x reference)
import functools

import jax
import jax.numpy as jnp
from jax import lax
from jax.experimental import pallas as pl
from jax.experimental.pallas import tpu as pltpu
from jax.experimental.pallas import tpu_sc as plsc

VOCAB = 100000
D = 128
TILE = 256
L = 16
TCB = 4096
NBUF = 4
S_TC = 18 * TCB


def _sc_matvec(weight, start):
    rows = VOCAB - start
    nt = -(-rows // TILE)
    info = plsc.get_sparse_core_info()
    nw = info.num_cores * info.num_subcores

    mesh = plsc.VectorSubcoreMesh(core_axis_name="c", subcore_axis_name="s")

    @functools.partial(
        pl.kernel,
        mesh=mesh,
        out_type=jax.ShapeDtypeStruct((rows,), jnp.float32),
        scratch_types=[
            pltpu.VMEM((2 * TILE, D), jnp.float32),
            pltpu.VMEM((2 * TILE,), jnp.float32),
            pltpu.SemaphoreType.DMA,
            pltpu.SemaphoreType.DMA,
        ],
        compiler_params=pltpu.CompilerParams(needs_layout_passes=False),
    )
    def k(w_hbm, out_hbm, wbuf, obuf, sem0, sem1):
        sems = (sem0, sem1)
        wid = lax.axis_index("s") * info.num_cores + lax.axis_index("c")
        lane = lax.iota(jnp.int32, L)
        lanef = lane.astype(jnp.float32)
        kvecs = [lanef + float(c * L) for c in range(D // L)]
        n_tiles = (nt - 1 - wid) // nw + 1

        def tile_base(i):
            return jnp.minimum((wid + nw * i) * TILE, rows - TILE)

        def in_copy(i, b, sem):
            return pltpu.make_async_copy(
                w_hbm.at[pl.ds(start + tile_base(i), TILE), :],
                wbuf.at[pl.ds(b * TILE, TILE), :],
                sem,
            )

        def compute(boff):
            def group_body(g, c2):
                row0 = boff + g * L

                def row_body(r, vec):
                    terms = [
                        wbuf[row0 + r, pl.ds(c * L, L)] * kvecs[c]
                        for c in range(D // L)
                    ]
                    while len(terms) > 1:
                        terms = [a + b2 for a, b2 in zip(terms[::2], terms[1::2])]
                    s = jnp.sum(terms[0])
                    return jnp.where(lane == r, s, vec)

                vec = lax.fori_loop(
                    0, L, row_body, jnp.zeros((L,), jnp.float32)
                )
                obuf[pl.ds(row0, L)] = vec
                return c2

            lax.fori_loop(0, TILE // L, group_body, 0)

        @pl.when(n_tiles > 0)
        def _():
            in_copy(0, 0, sem0).start()

        def body(i, carry):
            b = i % 2

            @pl.when((i + 1 < n_tiles) & (b == 0))
            def _():
                in_copy(i + 1, 1, sem1).start()

            @pl.when((i + 1 < n_tiles) & (b == 1))
            def _():
                in_copy(i + 1, 0, sem0).start()

            @pl.when(b == 0)
            def _():
                in_copy(i, 0, sem0).wait()

            @pl.when(b == 1)
            def _():
                in_copy(i, 1, sem1).wait()

            compute(b * TILE)
            pltpu.sync_copy(
                obuf.at[pl.ds(b * TILE, TILE)],
                out_hbm.at[pl.ds(tile_base(i), TILE)],
            )
            return carry

        lax.fori_loop(0, n_tiles, body, 0)

    return k(weight)


def _tc_matvec(weight, s_rows):
    steps = s_rows // TCB

    def body(w_hbm, o_ref, buf, sems):
        kv = lax.broadcasted_iota(jnp.int32, (1, D), 1).astype(jnp.float32)

        def chunk_copy(i, j):
            return pltpu.make_async_copy(
                w_hbm.at[pl.ds(i * TCB, TCB), :], buf.at[j], sems.at[j]
            )

        for j in range(min(NBUF, steps)):
            chunk_copy(j, j).start()

        def step(i, carry):
            j = i % NBUF
            chunk_copy(i, j).wait()
            res = lax.dot_general(
                kv,
                buf[j],
                (((1,), (1,)), ((), ())),
                preferred_element_type=jnp.float32,
            )
            o_ref[:, pl.ds(i * TCB, TCB)] = res

            @pl.when(i + NBUF < steps)
            def _():
                chunk_copy(i + NBUF, j).start()

            return carry

        lax.fori_loop(0, steps, step, 0)

    out = pl.pallas_call(
        body,
        in_specs=[pl.BlockSpec(memory_space=pltpu.HBM)],
        out_specs=pl.BlockSpec((1, s_rows), lambda: (0, 0)),
        out_shape=jax.ShapeDtypeStruct((1, s_rows), jnp.float32),
        scratch_shapes=[
            pltpu.VMEM((NBUF, TCB, D), jnp.float32),
            pltpu.SemaphoreType.DMA((NBUF,)),
        ],
    )(weight)
    return out.reshape(s_rows)


def kernel(x, weight):
    del x
    out_sc = _sc_matvec(weight, S_TC)
    out_tc = _tc_matvec(weight, S_TC)
    return jnp.concatenate([out_tc, out_sc])


# speedup vs baseline: 1.0897x; 1.0897x over previous
import functools

import jax
import jax.numpy as jnp
from jax import lax
from jax.experimental import pallas as pl
from jax.experimental.pallas import tpu as pltpu
from jax.experimental.pallas import tpu_sc as plsc

VOCAB = 100000
D = 128
TILE = 256
L = 16
TCB = 4096
NBUF = 4
S_TC = 17 * TCB


def _sc_matvec(weight, start):
    rows = VOCAB - start
    nt = -(-rows // TILE)
    info = plsc.get_sparse_core_info()
    nw = info.num_cores * info.num_subcores

    mesh = plsc.VectorSubcoreMesh(core_axis_name="c", subcore_axis_name="s")

    @functools.partial(
        pl.kernel,
        mesh=mesh,
        out_type=jax.ShapeDtypeStruct((rows,), jnp.float32),
        scratch_types=[
            pltpu.VMEM((2 * TILE, D), jnp.float32),
            pltpu.VMEM((2 * TILE,), jnp.float32),
            pltpu.SemaphoreType.DMA,
            pltpu.SemaphoreType.DMA,
        ],
        compiler_params=pltpu.CompilerParams(needs_layout_passes=False),
    )
    def k(w_hbm, out_hbm, wbuf, obuf, sem0, sem1):
        sems = (sem0, sem1)
        wid = lax.axis_index("s") * info.num_cores + lax.axis_index("c")
        lane = lax.iota(jnp.int32, L)
        lanef = lane.astype(jnp.float32)
        kvecs = [lanef + float(c * L) for c in range(D // L)]
        n_tiles = (nt - 1 - wid) // nw + 1

        def tile_base(i):
            return jnp.minimum((wid + nw * i) * TILE, rows - TILE)

        def in_copy(i, b, sem):
            return pltpu.make_async_copy(
                w_hbm.at[pl.ds(start + tile_base(i), TILE), :],
                wbuf.at[pl.ds(b * TILE, TILE), :],
                sem,
            )

        def compute(boff):
            def group_body(g, c2):
                row0 = boff + g * L

                def row_body(r, vec):
                    terms = [
                        wbuf[row0 + r, pl.ds(c * L, L)] * kvecs[c]
                        for c in range(D // L)
                    ]
                    while len(terms) > 1:
                        terms = [a + b2 for a, b2 in zip(terms[::2], terms[1::2])]
                    s = jnp.sum(terms[0])
                    return jnp.where(lane == r, s, vec)

                vec = lax.fori_loop(
                    0, L, row_body, jnp.zeros((L,), jnp.float32)
                )
                obuf[pl.ds(row0, L)] = vec
                return c2

            lax.fori_loop(0, TILE // L, group_body, 0)

        @pl.when(n_tiles > 0)
        def _():
            in_copy(0, 0, sem0).start()

        def body(i, carry):
            b = i % 2

            @pl.when((i + 1 < n_tiles) & (b == 0))
            def _():
                in_copy(i + 1, 1, sem1).start()

            @pl.when((i + 1 < n_tiles) & (b == 1))
            def _():
                in_copy(i + 1, 0, sem0).start()

            @pl.when(b == 0)
            def _():
                in_copy(i, 0, sem0).wait()

            @pl.when(b == 1)
            def _():
                in_copy(i, 1, sem1).wait()

            compute(b * TILE)
            pltpu.sync_copy(
                obuf.at[pl.ds(b * TILE, TILE)],
                out_hbm.at[pl.ds(tile_base(i), TILE)],
            )
            return carry

        lax.fori_loop(0, n_tiles, body, 0)

    return k(weight)


def _tc_matvec(weight, s_rows):
    steps = s_rows // TCB

    def body(w_hbm, o_ref, buf, sems):
        kv = lax.broadcasted_iota(jnp.int32, (1, D), 1).astype(jnp.float32)

        def chunk_copy(i, j):
            return pltpu.make_async_copy(
                w_hbm.at[pl.ds(i * TCB, TCB), :], buf.at[j], sems.at[j]
            )

        for j in range(min(NBUF, steps)):
            chunk_copy(j, j).start()

        def step(i, carry):
            j = i % NBUF
            chunk_copy(i, j).wait()
            res = lax.dot_general(
                kv,
                buf[j],
                (((1,), (1,)), ((), ())),
                preferred_element_type=jnp.float32,
            )
            o_ref[:, pl.ds(i * TCB, TCB)] = res

            @pl.when(i + NBUF < steps)
            def _():
                chunk_copy(i + NBUF, j).start()

            return carry

        lax.fori_loop(0, steps, step, 0)

    out = pl.pallas_call(
        body,
        in_specs=[pl.BlockSpec(memory_space=pltpu.HBM)],
        out_specs=pl.BlockSpec((1, s_rows), lambda: (0, 0)),
        out_shape=jax.ShapeDtypeStruct((1, s_rows), jnp.float32),
        scratch_shapes=[
            pltpu.VMEM((NBUF, TCB, D), jnp.float32),
            pltpu.SemaphoreType.DMA((NBUF,)),
        ],
    )(weight)
    return out.reshape(s_rows)


def kernel(x, weight):
    del x
    out_sc = _sc_matvec(weight, S_TC)
    out_tc = _tc_matvec(weight, S_TC)
    return jnp.concatenate([out_tc, out_sc])
